# baseline (speedup 1.0000x reference)
import functools

import jax
import jax.numpy as jnp
from jax import lax
from jax.experimental import pallas as pl
from jax.experimental.pallas import tpu as pltpu
from jax.experimental.pallas import tpu_sc as plsc

B = 16384
D = 64

_info = plsc.get_sparse_core_info()
NC, NS = _info.num_cores, _info.num_subcores
NW = NC * NS
BPW = B // NW
CHUNK = 128
NCH = BPW // CHUNK


def _sc_gather_body(utab, uidx, mtab, midx, uout, mout,
                    uidx_v, midx_v, urows, mrows, su, sm):
    wid = lax.axis_index("s") * NC + lax.axis_index("c")
    base = wid * BPW
    pltpu.sync_copy(uidx.at[wid], uidx_v)
    pltpu.sync_copy(midx.at[wid], midx_v)
    copies = []
    for j in range(NCH):
        copies.append(pltpu.async_copy(
            utab.at[uidx_v.at[j]], urows.at[pl.ds(j * CHUNK, CHUNK)], su))
        copies.append(pltpu.async_copy(
            mtab.at[midx_v.at[j]], mrows.at[pl.ds(j * CHUNK, CHUNK)], sm))
    for c in copies:
        c.wait()
    pltpu.sync_copy(urows, uout.at[pl.ds(base, BPW)])
    pltpu.sync_copy(mrows, mout.at[pl.ds(base, BPW)])


def _sc_gather(user_table, user_idx, movie_table, movie_idx):
    mesh = plsc.VectorSubcoreMesh(core_axis_name="c", subcore_axis_name="s")
    fn = functools.partial(
        pl.kernel, mesh=mesh,
        compiler_params=pltpu.CompilerParams(use_tc_tiling_on_sc=False),
        out_type=(jax.ShapeDtypeStruct((B, D), jnp.float32),
                  jax.ShapeDtypeStruct((B, D), jnp.float32)),
        scratch_types=[
            pltpu.VMEM((NCH, CHUNK), jnp.int32),
            pltpu.VMEM((NCH, CHUNK), jnp.int32),
            pltpu.VMEM((BPW, D), jnp.float32),
            pltpu.VMEM((BPW, D), jnp.float32),
            pltpu.SemaphoreType.DMA,
            pltpu.SemaphoreType.DMA,
        ],
    )(_sc_gather_body)
    uidx3 = user_idx.reshape(NW, NCH, CHUNK)
    midx3 = movie_idx.reshape(NW, NCH, CHUNK)
    return fn(user_table, uidx3, movie_table, midx3)


BLK = 2048


def _mlp_body(u_ref, m_ref, w1u_ref, w1m_ref, b1_ref, w2_ref, b2_ref,
              w3_ref, b3_ref, o_ref):
    h = jnp.dot(u_ref[...], w1u_ref[...], preferred_element_type=jnp.float32)
    h = h + jnp.dot(m_ref[...], w1m_ref[...],
                    preferred_element_type=jnp.float32)
    h = jnp.maximum(h + b1_ref[...], 0.0)
    h = jnp.dot(h, w2_ref[...], preferred_element_type=jnp.float32)
    h = jnp.maximum(h + b2_ref[...], 0.0)
    o_ref[...] = (jnp.dot(h, w3_ref[...], preferred_element_type=jnp.float32)
                  + b3_ref[...])


def _mlp(u_emb, m_emb, W1, b1, W2, b2, W3, b3):
    W1T = W1.T
    w1u = W1T[:D]
    w1m = W1T[D:]
    w2 = W2.T
    w3 = W3.T
    grid = (B // BLK,)
    return pl.pallas_call(
        _mlp_body,
        grid=grid,
        in_specs=[
            pl.BlockSpec((BLK, D), lambda i: (i, 0)),
            pl.BlockSpec((BLK, D), lambda i: (i, 0)),
            pl.BlockSpec((D, 128), lambda i: (0, 0)),
            pl.BlockSpec((D, 128), lambda i: (0, 0)),
            pl.BlockSpec((1, 128), lambda i: (0, 0)),
            pl.BlockSpec((128, D), lambda i: (0, 0)),
            pl.BlockSpec((1, D), lambda i: (0, 0)),
            pl.BlockSpec((D, 1), lambda i: (0, 0)),
            pl.BlockSpec((1, 1), lambda i: (0, 0)),
        ],
        out_specs=pl.BlockSpec((BLK, 1), lambda i: (i, 0)),
        out_shape=jax.ShapeDtypeStruct((B, 1), jnp.float32),
    )(u_emb, m_emb, w1u, w1m, b1.reshape(1, 128), w2, b2.reshape(1, D),
      w3, b3.reshape(1, 1))


def kernel(user, movie, user_table, movie_table, W1, b1, W2, b2, W3, b3):
    u_emb, m_emb = _sc_gather(user_table, user.astype(jnp.int32),
                              movie_table, movie.astype(jnp.int32))
    return _mlp(u_emb, m_emb, W1, b1, W2, b2, W3, b3)
